# baseline (speedup 1.0000x reference)
import functools

import jax
import jax.numpy as jnp
from jax import lax
from jax.experimental import pallas as pl
from jax.experimental.pallas import tpu as pltpu
from jax.experimental.pallas import tpu_sc as plsc

N = 50000
E = 800000
IN_F = 69
F1 = 96
HID = 128
OUT = 64

NC = 2
NT = 16

EPR = 128
E_PAD = 819200
ROWS = E_PAD // EPR
RPT = ROWS // NT
IDXR = 80
KG = 8
N_OUTER = RPT // IDXR
N_INNER = IDXR // KG

NAGG = 51200
RNG = 4
RROWS = NAGG // RNG
RTRASH = 8
RROWS_T = RROWS + RTRASH

RINGSZ = 11264
CAP = 224 * 256
PSTG = 16
EB = 64
NBUF = 4

STRIPE = NAGG // NT
RSTRIPE = RROWS // NT
WZR = 160
NWZ = RSTRIPE // WZR

RB = 512
GRID2 = NAGG // RB


def _deg_pass(srcd2d, dst2d):
    mesh = plsc.VectorSubcoreMesh(core_axis_name="c", subcore_axis_name="s")

    @functools.partial(
        pl.kernel,
        out_type=jax.ShapeDtypeStruct((NC, NAGG), jnp.float32),
        mesh=mesh,
        compiler_params=pltpu.CompilerParams(use_tc_tiling_on_sc=False),
        scratch_types=[
            pltpu.VMEM((IDXR, EPR), jnp.int32),
            pltpu.VMEM((EPR,), jnp.float32),
            pltpu.VMEM((STRIPE,), jnp.float32),
            pltpu.VMEM_SHARED((NAGG,), jnp.float32),
            pltpu.SemaphoreType.DMA,
        ],
    )
    def k(src_h, dst_h, out_h, idxv, ones_v, zflat, deg_sh, sem):
        c = lax.axis_index("c")
        s = lax.axis_index("s")

        zero16 = jnp.zeros((16,), jnp.float32)
        one16 = jnp.ones((16,), jnp.float32)

        def zfill(i, _):
            zflat[pl.ds(i * 16, 16)] = zero16
            return 0

        lax.fori_loop(0, STRIPE // 16, zfill, 0)
        for b in range(EPR // 16):
            ones_v[pl.ds(b * 16, 16)] = one16
        pltpu.sync_copy(zflat, deg_sh.at[pl.ds(s * STRIPE, STRIPE)])
        plsc.subcore_barrier()

        def process(idx_h):
            def outer(o, _):
                r0 = s * RPT + o * IDXR
                pltpu.sync_copy(idx_h.at[pl.ds(r0, IDXR)], idxv)

                def inner(g, _):
                    descs = [
                        pltpu.async_copy(
                            ones_v, deg_sh.at[idxv.at[g * KG + b]], sem, add=True
                        )
                        for b in range(KG)
                    ]
                    for d in descs:
                        d.wait()
                    return 0

                lax.fori_loop(0, N_INNER, inner, 0)
                return 0

            lax.fori_loop(0, N_OUTER, outer, 0)

        @pl.when(c == 0)
        def _():
            process(src_h)

        @pl.when(c == 1)
        def _():
            process(dst_h)

        plsc.subcore_barrier()
        pltpu.sync_copy(
            deg_sh.at[pl.ds(s * STRIPE, STRIPE)], out_h.at[c, pl.ds(s * STRIPE, STRIPE)]
        )

    return k(srcd2d, dst2d)


def _partition(src2d, dst2d):
    mesh = plsc.VectorSubcoreMesh(core_axis_name="c", subcore_axis_name="s")

    @functools.partial(
        pl.kernel,
        out_type=[
            jax.ShapeDtypeStruct((NC, NT, 2, CAP), jnp.int32),
            jax.ShapeDtypeStruct((NC, NT, 2, 16), jnp.int32),
        ],
        mesh=mesh,
        compiler_params=pltpu.CompilerParams(
            use_tc_tiling_on_sc=False, needs_layout_passes=False),
        scratch_types=[
            pltpu.VMEM((IDXR, EPR), jnp.int32),
            pltpu.VMEM((IDXR, EPR), jnp.int32),
            pltpu.VMEM((RINGSZ,), jnp.int32),
            pltpu.VMEM((RINGSZ,), jnp.int32),
            pltpu.VMEM((2, 16), jnp.int32),
        ],
    )
    def k(src_h, dst_h, plist_h, pcnt_h, sidx, didx, ring0, ring1, cntv):
        c = lax.axis_index("c")
        s = lax.axis_index("s")
        lo0 = (2 * c) * RROWS
        lo0s = lo0 * 65536
        los = RROWS * 65536
        tmask = jnp.ones((16,), jnp.bool_)
        trash16 = jnp.full((16,), RROWS * 65536, jnp.int32)

        rings = (ring0, ring1)

        def count(m):
            return jnp.max(plsc.all_reduce_population_count(m))

        def flush(ring, j, pos, fl):
            full = pos // 256

            def fk(kk, _):
                pltpu.sync_copy(
                    ring.at[pl.ds(kk * 256, 256)],
                    plist_h.at[c, s, j, pl.ds((fl + kk) * 256, 256)],
                )
                return 0

            lax.fori_loop(0, full, fk, 0)

            @pl.when(full > 0)
            def _():
                for t in range(16):
                    v = ring[pl.ds(full * 256 + 16 * t, 16)]
                    ring[pl.ds(16 * t, 16)] = v

            return pos - full * 256, fl + full

        def outer(o, carry):
            pos0, fl0, pos1, fl1 = carry
            r0 = s * RPT + o * IDXR
            pltpu.sync_copy(src_h.at[pl.ds(r0, IDXR)], sidx)
            pltpu.sync_copy(dst_h.at[pl.ds(r0, IDXR)], didx)

            def crow(r, carry2):
                pos0, pos1 = carry2
                for m in range(EPR // 16):
                    sv = sidx[r, pl.ds(m * 16, 16)]
                    dv = didx[r, pl.ds(m * 16, 16)]
                    dvs = dv * 65536
                    m0 = (dv >= lo0) & (dv < lo0 + RROWS)
                    p0 = (dvs - lo0s) | sv
                    plsc.store_compressed(ring0.at[pl.ds(pos0, 16)], p0, mask=m0)
                    pos0 = pos0 + count(m0)
                    m1 = (dv >= lo0 + RROWS) & (dv < lo0 + 2 * RROWS)
                    p1 = (dvs - lo0s - los) | sv
                    plsc.store_compressed(ring1.at[pl.ds(pos1, 16)], p1, mask=m1)
                    pos1 = pos1 + count(m1)
                return (pos0, pos1)

            pos0, pos1 = lax.fori_loop(0, IDXR, crow, (pos0, pos1))
            pos0, fl0 = flush(ring0, 0, pos0, fl0)
            pos1, fl1 = flush(ring1, 1, pos1, fl1)
            return (pos0, fl0, pos1, fl1)

        z = jnp.int32(0)
        pos0, fl0, pos1, fl1 = lax.fori_loop(0, N_OUTER, outer, (z, z, z, z))

        for j, (ring, pos, fl) in enumerate(((ring0, pos0, fl0), (ring1, pos1, fl1))):
            for t in range(16):
                plsc.store_compressed(ring.at[pl.ds(pos + 16 * t, 16)], trash16, mask=tmask)

            @pl.when(pos > 0)
            def _():
                pltpu.sync_copy(
                    ring.at[pl.ds(0, 256)],
                    plist_h.at[c, s, j, pl.ds(fl * 256, 256)],
                )

            n2 = fl + jnp.minimum(pos, 1)
            cntv[j, :] = jnp.full((16,), 1, jnp.int32) * n2

        pltpu.sync_copy(cntv, pcnt_h.at[c, s])

    return k(src2d, dst2d)


def _edge_pass(tab, plist, pcnt, zrows, F):
    mesh = plsc.VectorSubcoreMesh(core_axis_name="c", subcore_axis_name="s")

    @functools.partial(
        pl.kernel,
        out_type=jax.ShapeDtypeStruct((RNG, RROWS, F), jnp.float32),
        mesh=mesh,
        compiler_params=pltpu.CompilerParams(use_tc_tiling_on_sc=False),
        scratch_types=[
            pltpu.VMEM((PSTG * 256,), jnp.int32),
            pltpu.VMEM((NBUF, EB), jnp.int32),
            pltpu.VMEM((NBUF, EB), jnp.int32),
            pltpu.VMEM((NBUF, EB, F), jnp.float32),
            pltpu.VMEM((WZR, F), jnp.float32),
            pltpu.VMEM((2, 16), jnp.int32),
            pltpu.VMEM_SHARED((RROWS_T, F), jnp.float32),
        ] + [pltpu.SemaphoreType.DMA] * (2 * NBUF),
    )
    def k(tab_h, plist_h, pcnt_h, z_h, out_h,
          pbuf, sidxb, didxb, rowsb, zbuf, cntv, agg, *sems):
        gsems = sems[:NBUF]
        ssems = sems[NBUF:]
        c = lax.axis_index("c")
        s = lax.axis_index("s")

        pltpu.sync_copy(z_h, zbuf)
        pltpu.sync_copy(pcnt_h.at[c, s], cntv)

        def zero_stripe():
            def zloop(i, _):
                pltpu.sync_copy(zbuf, agg.at[pl.ds(s * RSTRIPE + i * WZR, WZR)])
                return 0

            lax.fori_loop(0, NWZ, zloop, 0)

            @pl.when(s == 0)
            def _():
                pltpu.sync_copy(zbuf.at[pl.ds(0, RTRASH)], agg.at[pl.ds(RROWS, RTRASH)])

        def writeout(q):
            def wloop(i, _):
                off = s * RSTRIPE + i * WZR
                pltpu.sync_copy(agg.at[pl.ds(off, WZR)], out_h.at[q, pl.ds(off, WZR)])
                return 0

            lax.fori_loop(0, NWZ, wloop, 0)

        def unpack(b, P):
            for mc in range(EB // 16):
                pk = pbuf[pl.ds(b * EB + mc * 16, 16)]
                sidxb[P, pl.ds(mc * 16, 16)] = pk & 0xFFFF
                didxb[P, pl.ds(mc * 16, 16)] = lax.shift_right_logical(pk, 16)

        def fire_g(b, P):
            return pltpu.async_copy(tab_h.at[sidxb.at[P]], rowsb.at[P], gsems[P])

        def fire_s(P):
            return pltpu.async_copy(
                rowsb.at[P], agg.at[didxb.at[P]], ssems[P], add=True
            )

        def process(r):
            n2 = cntv[r, pl.ds(0, 16)][0]
            nfull = n2 // PSTG

            def wait_s_rep(P):
                pltpu.make_async_copy(
                    rowsb.at[P], agg.at[didxb.at[P]], ssems[P]
                ).wait()

            def souter(t, _):
                base = t * PSTG
                pltpu.sync_copy(
                    plist_h.at[c, s, r, pl.ds(base * 256, PSTG * 256)], pbuf
                )
                nb = PSTG * 256 // EB
                gds = [None] * nb
                sds = [None] * nb
                for b in range(nb):
                    P = b % NBUF
                    if b >= NBUF:
                        wait_s_rep(P)
                    else:
                        @pl.when(t > 0)
                        def _(P=P):
                            wait_s_rep(P)
                    unpack(b, P)
                    gds[b] = fire_g(b, P)
                    if b >= 2:
                        Q = (b - 2) % NBUF
                        gds[b - 2].wait()
                        sds[b - 2] = fire_s(Q)
                for b in (nb - 2, nb - 1):
                    Q = b % NBUF
                    gds[b].wait()
                    sds[b] = fire_s(Q)
                return 0

            lax.fori_loop(0, nfull, souter, 0)

            @pl.when(nfull > 0)
            def _():
                for P in range(NBUF):
                    wait_s_rep(P)

            tbase = nfull * PSTG
            mm = n2 - tbase
            pltpu.sync_copy(
                plist_h.at[c, s, r, pl.ds(tbase * 256, PSTG * 256)], pbuf
            )

            def pair(jp, _):
                gp = [None] * NBUF
                for q in range(NBUF):
                    unpack(NBUF * jp + q, q)
                    gp[q] = fire_g(NBUF * jp + q, q)
                sp = [None] * NBUF
                for q in range(NBUF):
                    gp[q].wait()
                    sp[q] = fire_s(q)
                for q in range(NBUF):
                    sp[q].wait()
                return 0

            lax.fori_loop(0, mm, pair, 0)

        zero_stripe()
        plsc.subcore_barrier()
        process(0)
        plsc.subcore_barrier()
        writeout(2 * c)
        zero_stripe()
        plsc.subcore_barrier()
        process(1)
        plsc.subcore_barrier()
        writeout(2 * c + 1)

    return k(tab, plist, pcnt, zrows)


def _k2_body(deg_ref, x_ref, o_ref, ns_ref, nd_ref):
    d = jnp.transpose(deg_ref[...])
    ns = lax.rsqrt(jnp.maximum(d[:, 0:1], 1.0))
    nd = lax.rsqrt(jnp.maximum(d[:, 1:2], 1.0))
    o_ref[...] = x_ref[...] * ns
    ns_ref[...] = ns
    nd_ref[...] = nd


def _k2(deg, xp):
    return pl.pallas_call(
        _k2_body,
        grid=(GRID2,),
        in_specs=[
            pl.BlockSpec((NC, RB), lambda i: (0, i)),
            pl.BlockSpec((RB, F1), lambda i: (i, 0)),
        ],
        out_specs=[
            pl.BlockSpec((RB, F1), lambda i: (i, 0)),
            pl.BlockSpec((RB, 1), lambda i: (i, 0)),
            pl.BlockSpec((RB, 1), lambda i: (i, 0)),
        ],
        out_shape=[
            jax.ShapeDtypeStruct((NAGG, F1), jnp.float32),
            jax.ShapeDtypeStruct((NAGG, 1), jnp.float32),
            jax.ShapeDtypeStruct((NAGG, 1), jnp.float32),
        ],
    )(deg, xp)


def _k4_body(a_ref, ns_ref, nd_ref, w1_ref, w2_ref, p_ref):
    z = jnp.dot(a_ref[...] * nd_ref[...], w1_ref[...],
                preferred_element_type=jnp.float32)
    z = jnp.maximum(z, 0.0) * ns_ref[...]
    p_ref[...] = jnp.dot(z, w2_ref[...], preferred_element_type=jnp.float32)


def _k4(agg1, ns, nd, w1p, W2):
    return pl.pallas_call(
        _k4_body,
        grid=(GRID2,),
        in_specs=[
            pl.BlockSpec((RB, F1), lambda i: (i, 0)),
            pl.BlockSpec((RB, 1), lambda i: (i, 0)),
            pl.BlockSpec((RB, 1), lambda i: (i, 0)),
            pl.BlockSpec((F1, HID), lambda i: (0, 0)),
            pl.BlockSpec((HID, OUT), lambda i: (0, 0)),
        ],
        out_specs=pl.BlockSpec((RB, OUT), lambda i: (i, 0)),
        out_shape=jax.ShapeDtypeStruct((NAGG, OUT), jnp.float32),
    )(agg1, ns, nd, w1p, W2)


def _k6_body(a_ref, nd_ref, o_ref):
    i = pl.program_id(0)
    z = a_ref[...] * nd_ref[...]
    gid = lax.broadcasted_iota(jnp.int32, (RB, 1), 0) + i * RB
    z = jnp.where(gid < N, z, -3.0e38)
    m = jnp.max(z, axis=0, keepdims=True)

    @pl.when(i == 0)
    def _():
        o_ref[...] = m

    @pl.when(i > 0)
    def _():
        o_ref[...] = jnp.maximum(o_ref[...], m)

    @pl.when(i == GRID2 - 1)
    def _():
        o_ref[...] = jnp.maximum(o_ref[...], 0.0)


def _k6(agg2, nd):
    return pl.pallas_call(
        _k6_body,
        grid=(GRID2,),
        in_specs=[
            pl.BlockSpec((RB, OUT), lambda i: (i, 0)),
            pl.BlockSpec((RB, 1), lambda i: (i, 0)),
        ],
        out_specs=pl.BlockSpec((1, OUT), lambda i: (0, 0)),
        out_shape=jax.ShapeDtypeStruct((1, OUT), jnp.float32),
    )(agg2, nd)


def kernel(x, edge_index, W1, W2):
    ei = edge_index.astype(jnp.int32)
    src = ei[0]
    dst = ei[1]

    npad = E_PAD - E
    ar = jnp.arange(npad, dtype=jnp.int32)
    trash = N + (ar % 1024)
    src_deg2d = jnp.concatenate([src, trash]).reshape(ROWS, EPR)
    src_edge2d = jnp.concatenate([src, ar % 1024]).reshape(ROWS, EPR)
    dst2d = jnp.concatenate([dst, trash]).reshape(ROWS, EPR)

    xp = jnp.pad(x, ((0, NAGG - N), (0, F1 - IN_F)))
    w1p = jnp.pad(W1, ((0, F1 - IN_F), (0, 0)))
    z96 = jnp.zeros((WZR, F1), jnp.float32)
    z64 = jnp.zeros((WZR, OUT), jnp.float32)

    deg = _deg_pass(src_deg2d, dst2d)
    plist, pcnt = _partition(src_edge2d, dst2d)
    xn, ns, nd = _k2(deg, xp)
    agg1 = _edge_pass(xn, plist, pcnt, z96, F1).reshape(NAGG, F1)
    p = _k4(agg1, ns, nd, w1p, W2)
    agg2 = _edge_pass(p, plist, pcnt, z64, OUT).reshape(NAGG, OUT)
    return _k6(agg2, nd)
